# Initial kernel scaffold; baseline (speedup 1.0000x reference)
import jax
import jax.numpy as jnp
from jax import lax
from jax.experimental import pallas as pl
from jax.experimental.pallas import tpu as pltpu
from jax.experimental.pallas import tpu_sc as plsc

N = 100000
D = 128
G = 512

NC = 2
NS = 16
L = 16

DH = D // NC
SEGS = G // NS
ROWS_MAIN = 6248
CHUNK = 88
NCHUNK = ROWS_MAIN // CHUNK
TAIL = N - NS * ROWS_MAIN

_Z16 = jnp.zeros((L,), jnp.float32)
_O16 = jnp.ones((L,), jnp.float32)


def _body(embed_hbm, batch_hbm, w_hbm, out_hbm,
          dbuf, ibuf, dbuf2, ibuf2, ones_v, zbuf,
          abuf, wbuf, cbuf, obuf, acc_sh, cnt_sh):
  c = lax.axis_index("c")
  s = lax.axis_index("s")

  def zrow(i, _):
    for j in range(DH // L):
      zbuf[i, pl.ds(j * L, L)] = _Z16
    return 0
  lax.fori_loop(0, SEGS, zrow, 0)
  for j in range(SEGS // L):
    cbuf[pl.ds(j * L, L)] = _Z16
  for j in range(CHUNK // L):
    ones_v[pl.ds(j * L, L)] = _O16
  for j in range(CHUNK - (CHUNK // L) * L):
    ones_v[(CHUNK // L) * L + j] = 1.0

  pltpu.sync_copy(zbuf, acc_sh.at[pl.ds(s * SEGS, SEGS), :])
  pltpu.sync_copy(cbuf, cnt_sh.at[pl.ds(s * SEGS, SEGS)])
  plsc.subcore_barrier()

  def chunk(k, _):
    base = s * ROWS_MAIN + k * CHUNK
    pltpu.sync_copy(embed_hbm.at[pl.ds(base, CHUNK), pl.ds(c * DH, DH)], dbuf)
    pltpu.sync_copy(batch_hbm.at[pl.ds(base, CHUNK)], ibuf)
    pltpu.sync_copy(dbuf, acc_sh.at[ibuf], add=True)
    pltpu.sync_copy(ones_v, cnt_sh.at[ibuf], add=True)
    return 0
  lax.fori_loop(0, NCHUNK, chunk, 0)

  @pl.when(s == NS - 1)
  def _tail():
    base = NS * ROWS_MAIN
    pltpu.sync_copy(embed_hbm.at[pl.ds(base, TAIL), pl.ds(c * DH, DH)], dbuf2)
    pltpu.sync_copy(batch_hbm.at[pl.ds(base, TAIL)], ibuf2)
    pltpu.sync_copy(dbuf2, acc_sh.at[ibuf2], add=True)
    pltpu.sync_copy(ones_v.at[pl.ds(0, TAIL)], cnt_sh.at[ibuf2], add=True)

  plsc.subcore_barrier()

  pltpu.sync_copy(w_hbm.at[pl.ds(c * DH, DH)], wbuf)
  pltpu.sync_copy(acc_sh.at[pl.ds(s * SEGS, SEGS), :], abuf)
  pltpu.sync_copy(cnt_sh.at[pl.ds(s * SEGS, SEGS)], cbuf)

  def seg(g, _):
    v = _Z16
    for j in range(DH // L):
      v = v + abuf[g, pl.ds(j * L, L)] * wbuf[pl.ds(j * L, L)]
    tot = jnp.sum(v)
    cnt = jnp.maximum(cbuf[g], 1.0)
    obuf[g] = tot / cnt
    return 0
  lax.fori_loop(0, SEGS, seg, 0)

  pltpu.sync_copy(obuf, out_hbm.at[c, pl.ds(s * SEGS, SEGS)])


@jax.jit
def _pooled_linear(embed, batch_i32, w_flat):
  mesh = plsc.VectorSubcoreMesh(core_axis_name="c", subcore_axis_name="s",
                                num_cores=NC, num_subcores=NS)
  fn = pl.kernel(
      _body,
      out_type=jax.ShapeDtypeStruct((NC, G), jnp.float32),
      mesh=mesh,
      scratch_types=[
          pltpu.VMEM((CHUNK, DH), jnp.float32),
          pltpu.VMEM((CHUNK,), jnp.int32),
          pltpu.VMEM((TAIL, DH), jnp.float32),
          pltpu.VMEM((TAIL,), jnp.int32),
          pltpu.VMEM((CHUNK,), jnp.float32),
          pltpu.VMEM((SEGS, DH), jnp.float32),
          pltpu.VMEM((SEGS, DH), jnp.float32),
          pltpu.VMEM((DH,), jnp.float32),
          pltpu.VMEM((SEGS,), jnp.float32),
          pltpu.VMEM((SEGS,), jnp.float32),
          pltpu.VMEM_SHARED((G, DH), jnp.float32),
          pltpu.VMEM_SHARED((G,), jnp.float32),
      ],
  )
  return fn(embed, batch_i32, w_flat)


def kernel(embed, batch, W, b):
  partials = _pooled_linear(embed, batch.astype(jnp.int32), W.reshape(D))
  return (partials[0] + partials[1] + b[0]).reshape(G, 1)


# speedup vs baseline: 3.5496x; 3.5496x over previous
import jax
import jax.numpy as jnp
from jax import lax
from jax.experimental import pallas as pl
from jax.experimental.pallas import tpu as pltpu
from jax.experimental.pallas import tpu_sc as plsc

N = 100000
D = 128
G = 512

NC = 2
NS = 16
L = 16

DH = D // NC
SEGS = G // NS
ROWS_MAIN = 6248
CHUNK = 88
NCHUNK = ROWS_MAIN // CHUNK
TAIL = N - NS * ROWS_MAIN
ONES_LEN = 96

def _body(embed_hbm, batch_hbm, w_hbm, out_hbm,
          dbuf, ibuf, dbuf2, ibuf2, ones_v, zbuf,
          abuf, wbuf, cbuf, obuf, acc_sh, cnt_sh):
  c = lax.axis_index("c")
  s = lax.axis_index("s")
  _Z16 = jnp.zeros((L,), jnp.float32)
  _O16 = jnp.ones((L,), jnp.float32)

  def zrow(i, _):
    for j in range(DH // L):
      zbuf[i, pl.ds(j * L, L)] = _Z16
    return 0
  lax.fori_loop(0, SEGS, zrow, 0)
  for j in range(SEGS // L):
    cbuf[pl.ds(j * L, L)] = _Z16
  for j in range(ONES_LEN // L):
    ones_v[pl.ds(j * L, L)] = _O16

  pltpu.sync_copy(zbuf, acc_sh.at[pl.ds(s * SEGS, SEGS), :])
  pltpu.sync_copy(cbuf, cnt_sh.at[pl.ds(s * SEGS, SEGS)])
  plsc.subcore_barrier()

  def chunk(k, _):
    base = s * ROWS_MAIN + k * CHUNK
    pltpu.sync_copy(embed_hbm.at[pl.ds(base, CHUNK), pl.ds(c * DH, DH)], dbuf)
    pltpu.sync_copy(batch_hbm.at[pl.ds(base, CHUNK)], ibuf)
    pltpu.sync_copy(dbuf, acc_sh.at[ibuf], add=True)
    pltpu.sync_copy(ones_v.at[pl.ds(0, CHUNK)], cnt_sh.at[ibuf], add=True)
    return 0
  lax.fori_loop(0, NCHUNK, chunk, 0)

  @pl.when(s == NS - 1)
  def _tail():
    base = NS * ROWS_MAIN
    pltpu.sync_copy(embed_hbm.at[pl.ds(base, TAIL), pl.ds(c * DH, DH)], dbuf2)
    pltpu.sync_copy(batch_hbm.at[pl.ds(base, TAIL)], ibuf2)
    pltpu.sync_copy(dbuf2, acc_sh.at[ibuf2], add=True)
    pltpu.sync_copy(ones_v.at[pl.ds(0, TAIL)], cnt_sh.at[ibuf2], add=True)

  plsc.subcore_barrier()

  pltpu.sync_copy(w_hbm.at[pl.ds(c * DH, DH)], wbuf)
  pltpu.sync_copy(acc_sh.at[pl.ds(s * SEGS, SEGS), :], abuf)
  pltpu.sync_copy(cnt_sh.at[pl.ds(s * SEGS, SEGS)], cbuf)

  row_iota = lax.iota(jnp.int32, L)
  wvecs = [wbuf[pl.ds(j * L, L)] for j in range(DH // L)]
  for grp in range(SEGS // L):
    rows = row_iota + grp * L
    acc_v = _Z16
    for d in range(DH):
      col = jnp.full((L,), d, jnp.int32)
      w_s = wvecs[d // L][d % L]
      acc_v = acc_v + plsc.load_gather(abuf, [rows, col]) * w_s
    cnt_v = jnp.maximum(cbuf[pl.ds(grp * L, L)], 1.0)
    obuf[pl.ds(grp * L, L)] = acc_v / cnt_v

  pltpu.sync_copy(obuf, out_hbm.at[c, pl.ds(s * SEGS, SEGS)])


@jax.jit
def _pooled_linear(embed, batch_i32, w_flat):
  mesh = plsc.VectorSubcoreMesh(core_axis_name="c", subcore_axis_name="s",
                                num_cores=NC, num_subcores=NS)
  fn = pl.kernel(
      _body,
      out_type=jax.ShapeDtypeStruct((NC, G), jnp.float32),
      mesh=mesh,
      scratch_types=[
          pltpu.VMEM((CHUNK, DH), jnp.float32),
          pltpu.VMEM((CHUNK,), jnp.int32),
          pltpu.VMEM((TAIL, DH), jnp.float32),
          pltpu.VMEM((TAIL,), jnp.int32),
          pltpu.VMEM((ONES_LEN,), jnp.float32),
          pltpu.VMEM((SEGS, DH), jnp.float32),
          pltpu.VMEM((SEGS, DH), jnp.float32),
          pltpu.VMEM((DH,), jnp.float32),
          pltpu.VMEM((SEGS,), jnp.float32),
          pltpu.VMEM((SEGS,), jnp.float32),
          pltpu.VMEM_SHARED((G, DH), jnp.float32),
          pltpu.VMEM_SHARED((G,), jnp.float32),
      ],
      compiler_params=pltpu.CompilerParams(use_tc_tiling_on_sc=False,
                                           needs_layout_passes=False),
  )
  return fn(embed, batch_i32, w_flat)


def kernel(embed, batch, W, b):
  partials = _pooled_linear(embed, batch.astype(jnp.int32), W.reshape(D))
  return (partials[0] + partials[1] + b[0]).reshape(G, 1)


# speedup vs baseline: 6.4308x; 1.8117x over previous
import jax
import jax.numpy as jnp
from jax import lax
from jax.experimental import pallas as pl
from jax.experimental.pallas import tpu as pltpu
from jax.experimental.pallas import tpu_sc as plsc

N = 100000
D = 128
G = 512

NC = 2
NS = 16
L = 16

DH = D // NC
SEGS = G // NS
BR = 128
NB_FULL = N // BR
NB_MAIN = 48
NB_EXTRA = NB_FULL - NS * NB_MAIN
TAIL = N - NB_FULL * BR
TAIL_BASE = NB_FULL * BR


def _body(embed_hbm, batch_hbm, w_hbm, out_hbm,
          dbuf0, dbuf1, ibuf0, ibuf1, dbuf2, ibuf2, ones_v, zbuf,
          abuf, wbuf, cbuf, obuf, acc_sh, cnt_sh,
          lsem0, lsem1, ssem0, ssem1):
  c = lax.axis_index("c")
  s = lax.axis_index("s")
  _Z16 = jnp.zeros((L,), jnp.float32)
  _O16 = jnp.ones((L,), jnp.float32)

  b0 = jnp.where(s < NB_EXTRA, (NB_MAIN + 1) * s,
                 NB_EXTRA + NB_MAIN * s).astype(jnp.int32)

  def zrow(i, _):
    for j in range(DH // L):
      zbuf[i, pl.ds(j * L, L)] = _Z16
    return 0
  lax.fori_loop(0, SEGS, zrow, 0)
  for j in range(SEGS // L):
    cbuf[pl.ds(j * L, L)] = _Z16
  for j in range(BR // L):
    ones_v[pl.ds(j * L, L)] = _O16

  pltpu.sync_copy(zbuf, acc_sh.at[pl.ds(s * SEGS, SEGS), :])
  pltpu.sync_copy(cbuf, cnt_sh.at[pl.ds(s * SEGS, SEGS)])
  plsc.subcore_barrier()

  cols = pl.ds(c * DH, DH)

  def issue_load(k, dbuf, ibuf, lsem):
    base = (b0 + k) * BR
    pltpu.async_copy(embed_hbm.at[pl.ds(base, BR), cols], dbuf, lsem)
    pltpu.async_copy(batch_hbm.at[pl.ds(base, BR)], ibuf, lsem)

  def wait_load(k, dbuf, ibuf, lsem):
    base = (b0 + k) * BR
    pltpu.make_async_copy(embed_hbm.at[pl.ds(base, BR), cols], dbuf,
                          lsem).wait()
    pltpu.make_async_copy(batch_hbm.at[pl.ds(base, BR)], ibuf, lsem).wait()

  issue_load(0, dbuf0, ibuf0, lsem0)
  issue_load(1, dbuf1, ibuf1, lsem1)

  def step(j, _):
    for par, dbuf, ibuf, lsem, ssem in (
        (0, dbuf0, ibuf0, lsem0, ssem0),
        (1, dbuf1, ibuf1, lsem1, ssem1),
    ):
      k = 2 * j + par
      wait_load(k, dbuf, ibuf, lsem)
      sd = pltpu.async_copy(dbuf, acc_sh.at[ibuf], ssem, add=True)
      sc = pltpu.async_copy(ones_v, cnt_sh.at[ibuf], ssem, add=True)
      sd.wait()
      sc.wait()

      @pl.when(k + 2 < NB_MAIN)
      def _():
        issue_load(k + 2, dbuf, ibuf, lsem)
    return 0
  lax.fori_loop(0, NB_MAIN // 2, step, 0)

  @pl.when(s < NB_EXTRA)
  def _extra():
    base = (b0 + NB_MAIN) * BR
    pltpu.sync_copy(embed_hbm.at[pl.ds(base, BR), cols], dbuf0)
    pltpu.sync_copy(batch_hbm.at[pl.ds(base, BR)], ibuf0)
    pltpu.sync_copy(dbuf0, acc_sh.at[ibuf0], add=True)
    pltpu.sync_copy(ones_v, cnt_sh.at[ibuf0], add=True)

  @pl.when(s == NS - 1)
  def _tail():
    pltpu.sync_copy(embed_hbm.at[pl.ds(TAIL_BASE, TAIL), cols], dbuf2)
    pltpu.sync_copy(batch_hbm.at[pl.ds(TAIL_BASE, TAIL)], ibuf2)
    pltpu.sync_copy(dbuf2, acc_sh.at[ibuf2], add=True)
    pltpu.sync_copy(ones_v.at[pl.ds(0, TAIL)], cnt_sh.at[ibuf2], add=True)

  plsc.subcore_barrier()

  pltpu.sync_copy(w_hbm.at[pl.ds(c * DH, DH)], wbuf)
  pltpu.sync_copy(acc_sh.at[pl.ds(s * SEGS, SEGS), :], abuf)
  pltpu.sync_copy(cnt_sh.at[pl.ds(s * SEGS, SEGS)], cbuf)

  row_iota = lax.iota(jnp.int32, L)
  wvecs = [wbuf[pl.ds(j * L, L)] for j in range(DH // L)]
  for grp in range(SEGS // L):
    rows = row_iota + grp * L
    acc_v = _Z16
    for d in range(DH):
      col = jnp.full((L,), d, jnp.int32)
      w_s = wvecs[d // L][d % L]
      acc_v = acc_v + plsc.load_gather(abuf, [rows, col]) * w_s
    cnt_v = jnp.maximum(cbuf[pl.ds(grp * L, L)], 1.0)
    obuf[pl.ds(grp * L, L)] = acc_v / cnt_v

  pltpu.sync_copy(obuf, out_hbm.at[c, pl.ds(s * SEGS, SEGS)])


@jax.jit
def _pooled_linear(embed, batch_i32, w_flat):
  mesh = plsc.VectorSubcoreMesh(core_axis_name="c", subcore_axis_name="s",
                                num_cores=NC, num_subcores=NS)
  fn = pl.kernel(
      _body,
      out_type=jax.ShapeDtypeStruct((NC, G), jnp.float32),
      mesh=mesh,
      scratch_types=[
          pltpu.VMEM((BR, DH), jnp.float32),
          pltpu.VMEM((BR, DH), jnp.float32),
          pltpu.VMEM((BR,), jnp.int32),
          pltpu.VMEM((BR,), jnp.int32),
          pltpu.VMEM((TAIL, DH), jnp.float32),
          pltpu.VMEM((TAIL,), jnp.int32),
          pltpu.VMEM((BR,), jnp.float32),
          pltpu.VMEM((SEGS, DH), jnp.float32),
          pltpu.VMEM((SEGS, DH), jnp.float32),
          pltpu.VMEM((DH,), jnp.float32),
          pltpu.VMEM((SEGS,), jnp.float32),
          pltpu.VMEM((SEGS,), jnp.float32),
          pltpu.VMEM_SHARED((G, DH), jnp.float32),
          pltpu.VMEM_SHARED((G,), jnp.float32),
          pltpu.SemaphoreType.DMA,
          pltpu.SemaphoreType.DMA,
          pltpu.SemaphoreType.DMA,
          pltpu.SemaphoreType.DMA,
      ],
      compiler_params=pltpu.CompilerParams(use_tc_tiling_on_sc=False,
                                           needs_layout_passes=False),
  )
  return fn(embed, batch_i32, w_flat)


def kernel(embed, batch, W, b):
  partials = _pooled_linear(embed, batch.astype(jnp.int32), W.reshape(D))
  return (partials[0] + partials[1] + b[0]).reshape(G, 1)


# speedup vs baseline: 6.8399x; 1.0636x over previous
import jax
import jax.numpy as jnp
from jax import lax
from jax.experimental import pallas as pl
from jax.experimental.pallas import tpu as pltpu
from jax.experimental.pallas import tpu_sc as plsc

N = 100000
D = 128
G = 512

NC = 2
NS = 16
L = 16

DH = D // NC
SEGS = G // NS
BR = 128
NB_FULL = N // BR
NB_MAIN = 48
NB_EXTRA = NB_FULL - NS * NB_MAIN
TAIL = N - NB_FULL * BR
TAIL_BASE = NB_FULL * BR


def _body(embed_hbm, batch_hbm, w_hbm, out_hbm,
          dbuf0, dbuf1, dbuf2, dbuf3, ibuf0, ibuf1, ibuf2, ibuf3,
          dtail, itail, ones_v, zbuf,
          abuf, wbuf, cbuf, obuf, acc_sh, cnt_sh,
          lsem0, lsem1, lsem2, lsem3, ssem0, ssem1, ssem2, ssem3):
  c = lax.axis_index("c")
  s = lax.axis_index("s")
  _Z16 = jnp.zeros((L,), jnp.float32)
  _O16 = jnp.ones((L,), jnp.float32)

  b0 = jnp.where(s < NB_EXTRA, (NB_MAIN + 1) * s,
                 NB_EXTRA + NB_MAIN * s).astype(jnp.int32)

  def zrow(i, _):
    for j in range(DH // L):
      zbuf[i, pl.ds(j * L, L)] = _Z16
    return 0
  lax.fori_loop(0, SEGS, zrow, 0)
  for j in range(SEGS // L):
    cbuf[pl.ds(j * L, L)] = _Z16
  for j in range(BR // L):
    ones_v[pl.ds(j * L, L)] = _O16

  pltpu.sync_copy(zbuf, acc_sh.at[pl.ds(s * SEGS, SEGS), :])
  pltpu.sync_copy(cbuf, cnt_sh.at[pl.ds(s * SEGS, SEGS)])
  plsc.subcore_barrier()

  cols = pl.ds(c * DH, DH)

  def issue_load(k, dbuf, ibuf, lsem):
    base = (b0 + k) * BR
    pltpu.async_copy(embed_hbm.at[pl.ds(base, BR), cols], dbuf, lsem)
    pltpu.async_copy(batch_hbm.at[pl.ds(base, BR)], ibuf, lsem)

  def wait_load(k, dbuf, ibuf, lsem):
    base = (b0 + k) * BR
    pltpu.make_async_copy(embed_hbm.at[pl.ds(base, BR), cols], dbuf,
                          lsem).wait()
    pltpu.make_async_copy(batch_hbm.at[pl.ds(base, BR)], ibuf, lsem).wait()

  bufs = ((dbuf0, ibuf0, lsem0, ssem0),
          (dbuf1, ibuf1, lsem1, ssem1),
          (dbuf2, ibuf2, lsem2, ssem2),
          (dbuf3, ibuf3, lsem3, ssem3))

  def issue_scatter(dbuf, ibuf, ssem):
    pltpu.async_copy(dbuf, acc_sh.at[ibuf], ssem, add=True)
    pltpu.async_copy(ones_v, cnt_sh.at[ibuf], ssem, add=True)

  def wait_scatter(dbuf, ibuf, ssem):
    pltpu.make_async_copy(dbuf, acc_sh.at[ibuf], ssem).wait()
    pltpu.make_async_copy(ones_v, cnt_sh.at[ibuf], ssem).wait()

  issue_load(0, bufs[0][0], bufs[0][1], bufs[0][2])
  issue_load(1, bufs[1][0], bufs[1][1], bufs[1][2])

  def step(j, _):
    for par in range(4):
      dbuf, ibuf, lsem, ssem = bufs[par]
      m = 4 * j + par
      wait_load(m, dbuf, ibuf, lsem)
      issue_scatter(dbuf, ibuf, ssem)

      pbuf, pibuf, plsem, pssem = bufs[(par + 2) % 4]

      @pl.when(m + 2 < NB_MAIN)
      def _():
        @pl.when(m >= 2)
        def _():
          wait_scatter(pbuf, pibuf, pssem)
        issue_load(m + 2, pbuf, pibuf, plsem)
    return 0
  lax.fori_loop(0, NB_MAIN // 4, step, 0)

  for par in range(4):
    dbuf, ibuf, lsem, ssem = bufs[par]
    wait_scatter(dbuf, ibuf, ssem)

  @pl.when(s < NB_EXTRA)
  def _extra():
    base = (b0 + NB_MAIN) * BR
    pltpu.sync_copy(embed_hbm.at[pl.ds(base, BR), cols], dbuf0)
    pltpu.sync_copy(batch_hbm.at[pl.ds(base, BR)], ibuf0)
    pltpu.sync_copy(dbuf0, acc_sh.at[ibuf0], add=True)
    pltpu.sync_copy(ones_v, cnt_sh.at[ibuf0], add=True)

  @pl.when(s == NS - 1)
  def _tail():
    pltpu.sync_copy(embed_hbm.at[pl.ds(TAIL_BASE, TAIL), cols], dtail)
    pltpu.sync_copy(batch_hbm.at[pl.ds(TAIL_BASE, TAIL)], itail)
    pltpu.sync_copy(dtail, acc_sh.at[itail], add=True)
    pltpu.sync_copy(ones_v.at[pl.ds(0, TAIL)], cnt_sh.at[itail], add=True)

  plsc.subcore_barrier()

  pltpu.sync_copy(w_hbm.at[pl.ds(c * DH, DH)], wbuf)
  pltpu.sync_copy(acc_sh.at[pl.ds(s * SEGS, SEGS), :], abuf)
  pltpu.sync_copy(cnt_sh.at[pl.ds(s * SEGS, SEGS)], cbuf)

  row_iota = lax.iota(jnp.int32, L)
  wvecs = [wbuf[pl.ds(j * L, L)] for j in range(DH // L)]
  for grp in range(SEGS // L):
    rows = row_iota + grp * L
    acc_v = _Z16
    for d in range(DH):
      col = jnp.full((L,), d, jnp.int32)
      w_s = wvecs[d // L][d % L]
      acc_v = acc_v + plsc.load_gather(abuf, [rows, col]) * w_s
    cnt_v = jnp.maximum(cbuf[pl.ds(grp * L, L)], 1.0)
    obuf[pl.ds(grp * L, L)] = acc_v / cnt_v

  pltpu.sync_copy(obuf, out_hbm.at[c, pl.ds(s * SEGS, SEGS)])


@jax.jit
def _pooled_linear(embed, batch_i32, w_flat):
  mesh = plsc.VectorSubcoreMesh(core_axis_name="c", subcore_axis_name="s",
                                num_cores=NC, num_subcores=NS)
  fn = pl.kernel(
      _body,
      out_type=jax.ShapeDtypeStruct((NC, G), jnp.float32),
      mesh=mesh,
      scratch_types=[
          pltpu.VMEM((BR, DH), jnp.float32),
          pltpu.VMEM((BR, DH), jnp.float32),
          pltpu.VMEM((BR, DH), jnp.float32),
          pltpu.VMEM((BR, DH), jnp.float32),
          pltpu.VMEM((BR,), jnp.int32),
          pltpu.VMEM((BR,), jnp.int32),
          pltpu.VMEM((BR,), jnp.int32),
          pltpu.VMEM((BR,), jnp.int32),
          pltpu.VMEM((TAIL, DH), jnp.float32),
          pltpu.VMEM((TAIL,), jnp.int32),
          pltpu.VMEM((BR,), jnp.float32),
          pltpu.VMEM((SEGS, DH), jnp.float32),
          pltpu.VMEM((SEGS, DH), jnp.float32),
          pltpu.VMEM((DH,), jnp.float32),
          pltpu.VMEM((SEGS,), jnp.float32),
          pltpu.VMEM((SEGS,), jnp.float32),
          pltpu.VMEM_SHARED((G, DH), jnp.float32),
          pltpu.VMEM_SHARED((G,), jnp.float32),
          pltpu.SemaphoreType.DMA,
          pltpu.SemaphoreType.DMA,
          pltpu.SemaphoreType.DMA,
          pltpu.SemaphoreType.DMA,
          pltpu.SemaphoreType.DMA,
          pltpu.SemaphoreType.DMA,
          pltpu.SemaphoreType.DMA,
          pltpu.SemaphoreType.DMA,
      ],
      compiler_params=pltpu.CompilerParams(use_tc_tiling_on_sc=False,
                                           needs_layout_passes=False),
  )
  return fn(embed, batch_i32, w_flat)


def kernel(embed, batch, W, b):
  partials = _pooled_linear(embed, batch.astype(jnp.int32), W.reshape(D))
  return (partials[0] + partials[1] + b[0]).reshape(G, 1)
